# baseline (speedup 1.0000x reference)
import jax
import jax.numpy as jnp
from jax.experimental import pallas as pl
from jax.experimental.pallas import tpu as pltpu

_N, _INF, _OUTF = 8192, 1024, 1024
_CORES = 2
_CM = 512
_HALF = _N // _CORES
_NCHUNK = _HALF // _CM
_LOOK = 3
_YBUF = 3


def _manual_kernel(x_hbm, w_ref, b_ref, o_hbm, x_bufs, y_bufs, in_sems, out_sems):
    core = pl.program_id(0)
    base = core * _HALF
    w = w_ref[...]
    b = b_ref[...]

    def read(j):
        cp = pltpu.make_async_copy(
            x_hbm.at[pl.ds(base + j * _CM, _CM), :],
            x_bufs.at[j],
            in_sems.at[j],
        )
        cp.start()
        return cp

    in_copies = {}
    for j in range(_LOOK):
        in_copies[j] = read(j)

    out_copies = [None] * _YBUF
    for j in range(_NCHUNK):
        slot = j % _YBUF
        if out_copies[slot] is not None:
            out_copies[slot].wait()
        in_copies[j].wait()
        y = jnp.dot(x_bufs[j], w, preferred_element_type=jnp.float32)
        y_bufs[slot] = y + b
        cp = pltpu.make_async_copy(
            y_bufs.at[slot],
            o_hbm.at[pl.ds(base + j * _CM, _CM), :],
            out_sems.at[slot],
        )
        cp.start()
        out_copies[slot] = cp
        if j + _LOOK < _NCHUNK:
            in_copies[j + _LOOK] = read(j + _LOOK)
    for cp in out_copies:
        if cp is not None:
            cp.wait()


def kernel(x, w_fused, b_fused):
    y = pl.pallas_call(
        _manual_kernel,
        out_shape=jax.ShapeDtypeStruct((_N, _OUTF), jnp.float32),
        grid=(_CORES,),
        in_specs=[
            pl.BlockSpec(memory_space=pltpu.MemorySpace.HBM),
            pl.BlockSpec((_INF, _OUTF), lambda i: (0, 0)),
            pl.BlockSpec((1, _OUTF), lambda i: (0, 0)),
        ],
        out_specs=pl.BlockSpec(memory_space=pltpu.MemorySpace.HBM),
        scratch_shapes=[
            pltpu.VMEM((_NCHUNK, _CM, _INF), jnp.float32),
            pltpu.VMEM((_YBUF, _CM, _OUTF), jnp.float32),
            pltpu.SemaphoreType.DMA((_NCHUNK,)),
            pltpu.SemaphoreType.DMA((_YBUF,)),
        ],
        compiler_params=pltpu.CompilerParams(
            dimension_semantics=("parallel",)),
        cost_estimate=pl.CostEstimate(
            flops=2 * _N * _INF * _OUTF, transcendentals=0,
            bytes_accessed=4 * (_N * _INF + _N * _OUTF + _INF * _OUTF)),
    )(x, w_fused, b_fused)
    return y


# speedup vs baseline: 1.0207x; 1.0207x over previous
import jax
import jax.numpy as jnp
from jax.experimental import pallas as pl
from jax.experimental.pallas import tpu as pltpu

_LANES = 128
_SUBLANES = 8


def _round_up(x, m):
    return ((x + m - 1) // m) * m


def _fused_affine_kernel(x_ref, w_ref, b_ref, o_ref, wbf_ref):
    @pl.when(pl.program_id(0) == 0)
    def _():
        wbf_ref[...] = w_ref[...].astype(jnp.bfloat16)

    xb = x_ref[...].astype(jnp.bfloat16)
    y = jnp.dot(xb, wbf_ref[...], preferred_element_type=jnp.float32)
    o_ref[...] = y + b_ref[...]


def kernel(x, w_fused, b_fused):
    n, in_f = x.shape
    out_f = w_fused.shape[1]

    in_pad = _round_up(in_f, _LANES)
    out_pad = _round_up(out_f, _LANES)
    w_p = w_fused
    b_p = b_fused
    if in_pad != in_f or out_pad != out_f:
        w_p = jnp.zeros((in_pad, out_pad), jnp.float32).at[:in_f, :out_f].set(w_fused)
        b_p = jnp.zeros((1, out_pad), jnp.float32).at[:, :out_f].set(b_fused)

    x_p = x
    if in_pad != in_f:
        x_p = jnp.zeros((n, in_pad), jnp.float32).at[:, :in_f].set(x)

    tm = min(2048, _round_up(n, _SUBLANES))
    n_pad = _round_up(n, tm)
    if n_pad != n:
        x_p = jnp.zeros((n_pad, in_pad), x_p.dtype).at[:n, :].set(x_p)

    grid = (n_pad // tm,)
    y_pad = pl.pallas_call(
        _fused_affine_kernel,
        out_shape=jax.ShapeDtypeStruct((n_pad, out_pad), jnp.float32),
        grid=grid,
        in_specs=[
            pl.BlockSpec((tm, in_pad), lambda i: (i, 0)),
            pl.BlockSpec((in_pad, out_pad), lambda i: (0, 0)),
            pl.BlockSpec((1, out_pad), lambda i: (0, 0)),
        ],
        out_specs=pl.BlockSpec((tm, out_pad), lambda i: (i, 0)),
        scratch_shapes=[
            pltpu.VMEM((in_pad, out_pad), jnp.bfloat16),
        ],
        compiler_params=pltpu.CompilerParams(
            dimension_semantics=("arbitrary",)),
        cost_estimate=pl.CostEstimate(
            flops=2 * n_pad * in_pad * out_pad, transcendentals=0,
            bytes_accessed=4 * (n_pad * in_pad + n_pad * out_pad
                                + in_pad * out_pad)),
    )(x_p, w_p, b_p)

    if n_pad != n or out_pad != out_f:
        return y_pad[:n, :out_f]
    return y_pad
